# baseline (speedup 1.0000x reference)
import functools

import jax
import jax.numpy as jnp
from jax import lax
from jax.experimental import pallas as pl
from jax.experimental.pallas import tpu as pltpu
from jax.experimental.pallas import tpu_sc as plsc

_NBUF = 2
_CR = 8


def _make_gather(r, c, vocab, d):
    info = plsc.get_sparse_core_info()
    nc, ns = info.num_cores, info.num_subcores
    nw = nc * ns
    assert r % nw == 0
    rows_per_w = r // nw
    assert rows_per_w % _CR == 0
    n_chunks = rows_per_w // _CR
    assert n_chunks % _NBUF == 0 and n_chunks >= 2 * _NBUF

    mesh = plsc.VectorSubcoreMesh(core_axis_name="c", subcore_axis_name="s")

    @functools.partial(
        pl.kernel,
        out_type=jax.ShapeDtypeStruct((r, c, d), jnp.float32),
        mesh=mesh,
        scratch_types=[
            pltpu.VMEM((_CR, c), jnp.int32),
            pltpu.VMEM((_CR, c), jnp.int32),
            pltpu.VMEM((_CR, c, d), jnp.float32),
            pltpu.VMEM((_CR, c, d), jnp.float32),
            pltpu.SemaphoreType.DMA,
            pltpu.SemaphoreType.DMA,
            pltpu.SemaphoreType.DMA,
        ],
        compiler_params=pltpu.CompilerParams(use_tc_tiling_on_sc=False),
    )
    def gather_kernel(x_hbm, table_hbm, out_hbm, idx_v0, idx_v1, rows_v0,
                      rows_v1, idx_sem, gat_sem, out_sem):
        wid = lax.axis_index("s") * nc + lax.axis_index("c")
        base = wid * rows_per_w
        idx_bufs = (idx_v0, idx_v1)
        row_bufs = (rows_v0, rows_v1)

        def idx_start(i, b):
            pltpu.async_copy(x_hbm.at[pl.ds(base + i * _CR, _CR)],
                             idx_bufs[b], idx_sem)

        def idx_wait(b):
            pltpu.make_async_copy(x_hbm.at[pl.ds(base, _CR)],
                                  idx_bufs[b], idx_sem).wait()

        def gat_start(b):
            for j in range(_CR):
                pltpu.async_copy(table_hbm.at[idx_bufs[b].at[j]],
                                 row_bufs[b].at[j], gat_sem)

        def gat_wait(b):
            for j in range(_CR):
                pltpu.make_async_copy(table_hbm.at[idx_bufs[b].at[j]],
                                      row_bufs[b].at[j], gat_sem).wait()

        def out_start(i, b):
            pltpu.async_copy(row_bufs[b],
                             out_hbm.at[pl.ds(base + i * _CR, _CR)], out_sem)

        def out_wait(b):
            pltpu.make_async_copy(row_bufs[b],
                                  out_hbm.at[pl.ds(base, _CR)], out_sem).wait()

        idx_start(0, 0)
        idx_start(1, 1)
        for b in range(_NBUF):
            idx_wait(b)
            gat_start(b)
            gat_wait(b)
            idx_start(2 + b, b)
            out_start(b, b)

        def pair_body(io, carry):
            for b in range(_NBUF):
                i = io * _NBUF + b
                idx_wait(b)
                out_wait(b)
                gat_start(b)
                gat_wait(b)
                idx_start(i + 2, b)
                out_start(i, b)
            return carry

        lax.fori_loop(1, n_chunks // _NBUF - 1, pair_body, 0)

        for b in range(_NBUF):
            i = n_chunks - _NBUF + b
            idx_wait(b)
            out_wait(b)
            gat_start(b)
            gat_wait(b)
            out_start(i, b)
        for b in range(_NBUF):
            out_wait(b)

    return gather_kernel


def kernel(x, table):
    r, c = x.shape
    vocab, d = table.shape
    return _make_gather(r, c, vocab, d)(x, table)


# speedup vs baseline: 1.7020x; 1.7020x over previous
import functools

import jax
import jax.numpy as jnp
from jax import lax
from jax.experimental import pallas as pl
from jax.experimental.pallas import tpu as pltpu
from jax.experimental.pallas import tpu_sc as plsc

_JC = 2
_IW = 512
_L = 16


def _make_gather(r, c, vocab, d):
    info = plsc.get_sparse_core_info()
    nc, ns = info.num_cores, info.num_subcores
    nw = nc * ns
    assert r % nw == 0 and r // nw == _IW and d == _L
    assert c % _JC == 0
    n_chunks = c // _JC

    mesh = plsc.VectorSubcoreMesh(core_axis_name="c", subcore_axis_name="s")

    @functools.partial(
        pl.kernel,
        out_type=jax.ShapeDtypeStruct((c, d, r), jnp.float32),
        mesh=mesh,
        scratch_types=[
            pltpu.VMEM((_JC, _IW), jnp.int32),
            pltpu.VMEM((_JC, _IW), jnp.int32),
            pltpu.VMEM((_JC, _IW, d), jnp.float32),
            pltpu.VMEM((_JC, _IW, d), jnp.float32),
            pltpu.VMEM((_JC, d, _IW), jnp.float32),
            pltpu.VMEM((_JC, d, _IW), jnp.float32),
            pltpu.SemaphoreType.DMA,
            pltpu.SemaphoreType.DMA,
            pltpu.SemaphoreType.DMA,
        ],
        compiler_params=pltpu.CompilerParams(
            use_tc_tiling_on_sc=False, needs_layout_passes=False),
    )
    def gather_kernel(xt_hbm, table_hbm, out_hbm, idx_v0, idx_v1, g_v0, g_v1,
                      t_v0, t_v1, idx_sem, gat_sem, out_sem):
        wid = lax.axis_index("s") * nc + lax.axis_index("c")
        i0 = wid * _IW
        idx_bufs = (idx_v0, idx_v1)
        g_bufs = (g_v0, g_v1)
        t_bufs = (t_v0, t_v1)

        def idx_start(ch, b):
            pltpu.async_copy(
                xt_hbm.at[pl.ds(ch * _JC, _JC), pl.ds(i0, _IW)],
                idx_bufs[b], idx_sem)

        def idx_wait(b):
            pltpu.make_async_copy(
                xt_hbm.at[pl.ds(0, _JC), pl.ds(i0, _IW)],
                idx_bufs[b], idx_sem).wait()

        def gat_start(b):
            for jj in range(_JC):
                pltpu.async_copy(table_hbm.at[idx_bufs[b].at[jj]],
                                 g_bufs[b].at[jj], gat_sem)

        def gat_wait(b):
            for jj in range(_JC):
                pltpu.make_async_copy(table_hbm.at[idx_bufs[b].at[jj]],
                                      g_bufs[b].at[jj], gat_sem).wait()

        def out_start(ch, b):
            pltpu.async_copy(
                t_bufs[b],
                out_hbm.at[pl.ds(ch * _JC, _JC), :, pl.ds(i0, _IW)], out_sem)

        def out_wait(b):
            pltpu.make_async_copy(
                t_bufs[b],
                out_hbm.at[pl.ds(0, _JC), :, pl.ds(i0, _IW)], out_sem).wait()

        def transpose_chunk(b):
            g_v, t_v = g_bufs[b], t_bufs[b]
            d_ids = [jnp.full((_L,), dd, jnp.int32) for dd in range(d)]

            def body(k, carry):
                ib = k * _L
                ii = ib + lax.iota(jnp.int32, _L)
                for jj in range(_JC):
                    for dd in range(d):
                        vals = plsc.load_gather(g_v.at[jj], [ii, d_ids[dd]])
                        t_v[jj, dd, pl.ds(ib, _L)] = vals
                return carry

            lax.fori_loop(0, _IW // _L, body, 0)

        idx_start(0, 0)
        idx_start(1, 1)
        idx_wait(0)
        gat_start(0)
        for ch in (0, 1):
            b, b2 = ch % 2, (ch + 1) % 2
            gat_wait(b)
            if ch + 1 < n_chunks:
                idx_wait(b2)
                gat_start(b2)
            idx_start(ch + 2, b)
            transpose_chunk(b)
            out_start(ch, b)

        def pair_body(io, carry):
            for b in range(2):
                ch = io * 2 + b
                b2 = (b + 1) % 2
                gat_wait(b)
                idx_wait(b2)
                gat_start(b2)
                idx_start(ch + 2, b)
                out_wait(b)
                transpose_chunk(b)
                out_start(ch, b)
            return carry

        lax.fori_loop(1, n_chunks // 2 - 1, pair_body, 0)

        for ch in (n_chunks - 2, n_chunks - 1):
            b, b2 = ch % 2, (ch + 1) % 2
            gat_wait(b)
            if ch + 1 < n_chunks:
                idx_wait(b2)
                gat_start(b2)
            out_wait(b)
            transpose_chunk(b)
            out_start(ch, b)
        for b in range(2):
            out_wait(b)

    return gather_kernel


def kernel(x, table):
    r, c = x.shape
    vocab, d = table.shape
    xt = jnp.transpose(x)
    out_t = _make_gather(r, c, vocab, d)(xt, table)
    return jnp.transpose(out_t, (2, 0, 1))


# speedup vs baseline: 2.3654x; 1.3898x over previous
import functools

import jax
import jax.numpy as jnp
from jax import lax
from jax.experimental import pallas as pl
from jax.experimental.pallas import tpu as pltpu
from jax.experimental.pallas import tpu_sc as plsc

_JC = 2
_IW = 512
_L = 16


def _make_gather(r, c, vocab, d):
    info = plsc.get_sparse_core_info()
    nc, ns = info.num_cores, info.num_subcores
    nw = nc * ns
    assert r % nw == 0 and r // nw == _IW and d == _L
    assert c % _JC == 0
    n_chunks = c // _JC

    mesh = plsc.VectorSubcoreMesh(core_axis_name="c", subcore_axis_name="s")

    @functools.partial(
        pl.kernel,
        out_type=jax.ShapeDtypeStruct((c, d, r), jnp.float32),
        mesh=mesh,
        scratch_types=[
            pltpu.VMEM((_JC, _IW), jnp.int32),
            pltpu.VMEM((_JC, _IW), jnp.int32),
            pltpu.VMEM((_JC, _IW, d), jnp.float32),
            pltpu.VMEM((_JC, _IW, d), jnp.float32),
            pltpu.VMEM((_JC, d, _IW), jnp.float32),
            pltpu.VMEM((_JC, d, _IW), jnp.float32),
            pltpu.SemaphoreType.DMA,
            pltpu.SemaphoreType.DMA,
            pltpu.SemaphoreType.DMA,
        ],
        compiler_params=pltpu.CompilerParams(
            use_tc_tiling_on_sc=False, needs_layout_passes=False),
    )
    def gather_kernel(xt_hbm, table_hbm, out_hbm, idx_v0, idx_v1, g_v0, g_v1,
                      t_v0, t_v1, idx_sem, gat_sem, out_sem):
        wid = lax.axis_index("s") * nc + lax.axis_index("c")
        i0 = wid * _IW
        idx_bufs = (idx_v0, idx_v1)
        g_bufs = (g_v0, g_v1)
        t_bufs = (t_v0, t_v1)

        def idx_start(ch, b):
            pltpu.async_copy(
                xt_hbm.at[pl.ds(ch * _JC, _JC), pl.ds(i0, _IW)],
                idx_bufs[b], idx_sem)

        def idx_wait(b):
            pltpu.make_async_copy(
                xt_hbm.at[pl.ds(0, _JC), pl.ds(i0, _IW)],
                idx_bufs[b], idx_sem).wait()

        def gat_start(b):
            for jj in range(_JC):
                pltpu.async_copy(table_hbm.at[idx_bufs[b].at[jj]],
                                 g_bufs[b].at[jj], gat_sem)

        def gat_wait(b):
            for jj in range(_JC):
                pltpu.make_async_copy(table_hbm.at[idx_bufs[b].at[jj]],
                                      g_bufs[b].at[jj], gat_sem).wait()

        def out_start(ch, b):
            pltpu.async_copy(
                t_bufs[b],
                out_hbm.at[pl.ds(ch * _JC, _JC), :, pl.ds(i0, _IW)], out_sem)

        def out_wait(b):
            pltpu.make_async_copy(
                t_bufs[b],
                out_hbm.at[pl.ds(0, _JC), :, pl.ds(i0, _IW)], out_sem).wait()

        lanes = lax.iota(jnp.int32, _L)
        rots = [(lanes + r) & (_L - 1) for r in range(_L)]

        def transpose_chunk(b):
            g_v, t_v = g_bufs[b], t_bufs[b]

            def body(k, carry):
                ii = k * _L + lanes
                for jj in range(_JC):
                    for r in range(_L):
                        vals = plsc.load_gather(g_v.at[jj], [ii, rots[r]])
                        plsc.store_scatter(t_v.at[jj], [rots[r], ii], vals)
                return carry

            lax.fori_loop(0, _IW // _L, body, 0)

        idx_start(0, 0)
        idx_start(1, 1)
        idx_wait(0)
        gat_start(0)
        for ch in (0, 1):
            b, b2 = ch % 2, (ch + 1) % 2
            gat_wait(b)
            if ch + 1 < n_chunks:
                idx_wait(b2)
                gat_start(b2)
            idx_start(ch + 2, b)
            transpose_chunk(b)
            out_start(ch, b)

        def pair_body(io, carry):
            for b in range(2):
                ch = io * 2 + b
                b2 = (b + 1) % 2
                gat_wait(b)
                idx_wait(b2)
                gat_start(b2)
                idx_start(ch + 2, b)
                out_wait(b)
                transpose_chunk(b)
                out_start(ch, b)
            return carry

        lax.fori_loop(1, n_chunks // 2 - 1, pair_body, 0)

        for ch in (n_chunks - 2, n_chunks - 1):
            b, b2 = ch % 2, (ch + 1) % 2
            gat_wait(b)
            if ch + 1 < n_chunks:
                idx_wait(b2)
                gat_start(b2)
            out_wait(b)
            transpose_chunk(b)
            out_start(ch, b)
        for b in range(2):
            out_wait(b)

    return gather_kernel


def kernel(x, table):
    r, c = x.shape
    vocab, d = table.shape
    xt = jnp.transpose(x)
    out_t = _make_gather(r, c, vocab, d)(xt, table)
    return jnp.transpose(out_t, (2, 0, 1))


# speedup vs baseline: 2.9767x; 1.2584x over previous
import functools

import jax
import jax.numpy as jnp
from jax import lax
from jax.experimental import pallas as pl
from jax.experimental.pallas import tpu as pltpu
from jax.experimental.pallas import tpu_sc as plsc

_JC = 2
_IW = 512
_L = 16


def _make_gather(r, c, vocab, d):
    info = plsc.get_sparse_core_info()
    nc, ns = info.num_cores, info.num_subcores
    nw = nc * ns
    assert r % nw == 0 and r // nw == _IW and d == _L
    assert c % _JC == 0
    n_chunks = c // _JC

    mesh = plsc.VectorSubcoreMesh(core_axis_name="c", subcore_axis_name="s")

    nb = r // 128
    @functools.partial(
        pl.kernel,
        out_type=jax.ShapeDtypeStruct((c, d // 8, nb, 8, 128), jnp.float32),
        mesh=mesh,
        scratch_types=[
            pltpu.VMEM((_JC, _IW), jnp.int32),
            pltpu.VMEM((_JC, _IW), jnp.int32),
            pltpu.VMEM((_JC, _IW, d), jnp.float32),
            pltpu.VMEM((_JC, _IW, d), jnp.float32),
            pltpu.VMEM((_JC, d // 8, _IW // 128, 8, 128), jnp.float32),
            pltpu.VMEM((_JC, d // 8, _IW // 128, 8, 128), jnp.float32),
            pltpu.SemaphoreType.DMA,
            pltpu.SemaphoreType.DMA,
            pltpu.SemaphoreType.DMA,
        ],
        compiler_params=pltpu.CompilerParams(
            use_tc_tiling_on_sc=False, needs_layout_passes=False),
    )
    def gather_kernel(xt_hbm, table_hbm, out_hbm, idx_v0, idx_v1, g_v0, g_v1,
                      t_v0, t_v1, idx_sem, gat_sem, out_sem):
        wid = lax.axis_index("s") * nc + lax.axis_index("c")
        i0 = wid * _IW
        idx_bufs = (idx_v0, idx_v1)
        g_bufs = (g_v0, g_v1)
        t_bufs = (t_v0, t_v1)

        def idx_start(ch, b):
            pltpu.async_copy(
                xt_hbm.at[pl.ds(ch * _JC, _JC), pl.ds(i0, _IW)],
                idx_bufs[b], idx_sem)

        def idx_wait(b):
            pltpu.make_async_copy(
                xt_hbm.at[pl.ds(0, _JC), pl.ds(i0, _IW)],
                idx_bufs[b], idx_sem).wait()

        def gat_start(b):
            for jj in range(_JC):
                pltpu.async_copy(table_hbm.at[idx_bufs[b].at[jj]],
                                 g_bufs[b].at[jj], gat_sem)

        def gat_wait(b):
            for jj in range(_JC):
                pltpu.make_async_copy(table_hbm.at[idx_bufs[b].at[jj]],
                                      g_bufs[b].at[jj], gat_sem).wait()

        ib0 = wid * (_IW // 128)

        def out_start(ch, b):
            pltpu.async_copy(
                t_bufs[b],
                out_hbm.at[pl.ds(ch * _JC, _JC), :, pl.ds(ib0, _IW // 128)],
                out_sem)

        def out_wait(b):
            pltpu.make_async_copy(
                t_bufs[b],
                out_hbm.at[pl.ds(0, _JC), :, pl.ds(ib0, _IW // 128)],
                out_sem).wait()

        lanes = lax.iota(jnp.int32, _L)
        rots = [(lanes + r) & (_L - 1) for r in range(_L)]
        rot_db = [rv >> 3 for rv in rots]
        rot_rr = [rv & 7 for rv in rots]

        def transpose_chunk(b):
            g_v, t_v = g_bufs[b], t_bufs[b]

            def body(k, carry):
                ii = k * _L + lanes
                ibl = ii >> 7
                cc = ii & 127
                for jj in range(_JC):
                    for r in range(_L):
                        vals = plsc.load_gather(g_v.at[jj], [ii, rots[r]])
                        plsc.store_scatter(
                            t_v.at[jj], [rot_db[r], ibl, rot_rr[r], cc], vals)
                return carry

            lax.fori_loop(0, _IW // _L, body, 0)

        idx_start(0, 0)
        idx_start(1, 1)
        idx_wait(0)
        gat_start(0)
        for ch in (0, 1):
            b, b2 = ch % 2, (ch + 1) % 2
            gat_wait(b)
            if ch + 1 < n_chunks:
                idx_wait(b2)
                gat_start(b2)
            idx_start(ch + 2, b)
            transpose_chunk(b)
            out_start(ch, b)

        def pair_body(io, carry):
            for b in range(2):
                ch = io * 2 + b
                b2 = (b + 1) % 2
                gat_wait(b)
                idx_wait(b2)
                gat_start(b2)
                idx_start(ch + 2, b)
                out_wait(b)
                transpose_chunk(b)
                out_start(ch, b)
            return carry

        lax.fori_loop(1, n_chunks // 2 - 1, pair_body, 0)

        for ch in (n_chunks - 2, n_chunks - 1):
            b, b2 = ch % 2, (ch + 1) % 2
            gat_wait(b)
            if ch + 1 < n_chunks:
                idx_wait(b2)
                gat_start(b2)
            out_wait(b)
            transpose_chunk(b)
            out_start(ch, b)
        for b in range(2):
            out_wait(b)

    return gather_kernel


def kernel(x, table):
    r, c = x.shape
    vocab, d = table.shape
    xt = jnp.transpose(x)
    out5 = _make_gather(r, c, vocab, d)(xt, table)
    return jnp.transpose(out5, (2, 4, 0, 1, 3)).reshape(r, c, d)


# speedup vs baseline: 4.5743x; 1.5367x over previous
import functools

import jax
import jax.numpy as jnp
from jax import lax
from jax.experimental import pallas as pl
from jax.experimental.pallas import tpu as pltpu
from jax.experimental.pallas import tpu_sc as plsc

_JC = 2
_IW = 512
_L = 16


def _make_gather(r, c, vocab, d):
    info = plsc.get_sparse_core_info()
    nc, ns = info.num_cores, info.num_subcores
    nw = nc * ns
    assert r % nw == 0 and r // nw == _IW and d == _L
    assert c % _JC == 0
    n_chunks = c // _JC

    mesh = plsc.VectorSubcoreMesh(core_axis_name="c", subcore_axis_name="s")

    nb = r // 128
    @functools.partial(
        pl.kernel,
        out_type=jax.ShapeDtypeStruct((c, d // 8, nb, 8, 128), jnp.float32),
        mesh=mesh,
        scratch_types=[
            pltpu.VMEM((_JC, _IW), jnp.int32),
            pltpu.VMEM((_JC, _IW), jnp.int32),
            pltpu.VMEM((_JC, _IW, d), jnp.float32),
            pltpu.VMEM((_JC, _IW, d), jnp.float32),
            pltpu.VMEM((_JC, d // 8, _IW // 128, 8, 128), jnp.float32),
            pltpu.VMEM((_JC, d // 8, _IW // 128, 8, 128), jnp.float32),
            pltpu.SemaphoreType.DMA,
            pltpu.SemaphoreType.DMA,
            pltpu.SemaphoreType.DMA,
        ],
        compiler_params=pltpu.CompilerParams(
            use_tc_tiling_on_sc=False, needs_layout_passes=False),
    )
    def gather_kernel(xt_hbm, table_hbm, out_hbm, idx_v0, idx_v1, g_v0, g_v1,
                      t_v0, t_v1, idx_sem, gat_sem, out_sem):
        wid = lax.axis_index("s") * nc + lax.axis_index("c")
        i0 = wid * _IW
        idx_bufs = (idx_v0, idx_v1)
        g_bufs = (g_v0, g_v1)
        t_bufs = (t_v0, t_v1)

        def idx_start(ch, b):
            pltpu.async_copy(
                xt_hbm.at[pl.ds(ch * _JC, _JC), pl.ds(i0, _IW)],
                idx_bufs[b], idx_sem)

        def idx_wait(b):
            pltpu.make_async_copy(
                xt_hbm.at[pl.ds(0, _JC), pl.ds(i0, _IW)],
                idx_bufs[b], idx_sem).wait()

        def gat_start(b):
            for jj in range(_JC):
                pltpu.async_copy(table_hbm.at[idx_bufs[b].at[jj]],
                                 g_bufs[b].at[jj], gat_sem)

        def gat_wait(b):
            for jj in range(_JC):
                pltpu.make_async_copy(table_hbm.at[idx_bufs[b].at[jj]],
                                      g_bufs[b].at[jj], gat_sem).wait()

        ib0 = wid * (_IW // 128)

        def out_start(ch, b):
            pltpu.async_copy(
                t_bufs[b],
                out_hbm.at[pl.ds(ch * _JC, _JC), :, pl.ds(ib0, _IW // 128)],
                out_sem)

        def out_wait(b):
            pltpu.make_async_copy(
                t_bufs[b],
                out_hbm.at[pl.ds(0, _JC), :, pl.ds(ib0, _IW // 128)],
                out_sem).wait()

        lanes = lax.iota(jnp.int32, _L)
        rots = [(lanes + r) & (_L - 1) for r in range(_L)]
        rot_db = [rv >> 3 for rv in rots]
        rot_rr = [rv & 7 for rv in rots]

        def transpose_chunk(b):
            g_v, t_v = g_bufs[b], t_bufs[b]

            def body(k, carry):
                ii = k * _L + lanes
                ibl = ii >> 7
                cc = ii & 127
                for jj in range(_JC):
                    for r in range(_L):
                        vals = plsc.load_gather(g_v.at[jj], [ii, rots[r]])
                        plsc.store_scatter(
                            t_v.at[jj], [rot_db[r], ibl, rot_rr[r], cc], vals)
                return carry

            lax.fori_loop(0, _IW // _L, body, 0)

        idx_start(0, 0)
        idx_start(1, 1)
        idx_wait(0)
        gat_start(0)
        for ch in (0, 1):
            b, b2 = ch % 2, (ch + 1) % 2
            gat_wait(b)
            if ch + 1 < n_chunks:
                idx_wait(b2)
                gat_start(b2)
            idx_start(ch + 2, b)
            transpose_chunk(b)
            out_start(ch, b)

        def pair_body(io, carry):
            for b in range(2):
                ch = io * 2 + b
                b2 = (b + 1) % 2
                gat_wait(b)
                idx_wait(b2)
                gat_start(b2)
                idx_start(ch + 2, b)
                out_wait(b)
                transpose_chunk(b)
                out_start(ch, b)
            return carry

        lax.fori_loop(1, n_chunks // 2 - 1, pair_body, 0)

        for ch in (n_chunks - 2, n_chunks - 1):
            b, b2 = ch % 2, (ch + 1) % 2
            gat_wait(b)
            if ch + 1 < n_chunks:
                idx_wait(b2)
                gat_start(b2)
            out_wait(b)
            transpose_chunk(b)
            out_start(ch, b)
        for b in range(2):
            out_wait(b)

    return gather_kernel


def _make_detile(vocab, d):
    info = plsc.get_sparse_core_info()
    nc, ns = info.num_cores, info.num_subcores
    nw = nc * ns
    nb_full = vocab // 128
    tail = vocab - nb_full * 128
    bpc = 4
    per_w = nb_full // bpc // nw
    extra = nb_full - per_w * bpc * nw
    cw = bpc * 128

    mesh = plsc.VectorSubcoreMesh(core_axis_name="c", subcore_axis_name="s")

    @functools.partial(
        pl.kernel,
        out_type=jax.ShapeDtypeStruct((vocab * d // 128, 128), jnp.float32),
        mesh=mesh,
        scratch_types=[
            pltpu.VMEM((d, cw), jnp.float32),
            pltpu.VMEM((d, cw), jnp.float32),
            pltpu.VMEM((cw // 8, 128), jnp.float32),
            pltpu.VMEM((cw // 8, 128), jnp.float32),
            pltpu.VMEM((d, 128), jnp.float32),
            pltpu.VMEM((16, 128), jnp.float32),
            pltpu.VMEM((tail * d // 128 if tail else 8, 128), jnp.float32),
            pltpu.SemaphoreType.DMA,
            pltpu.SemaphoreType.DMA,
        ],
        compiler_params=pltpu.CompilerParams(
            use_tc_tiling_on_sc=True, needs_layout_passes=False),
    )
    def detile_kernel(tt_hbm, tail_hbm, out_hbm, g_v0, g_v1, o_v0, o_v1,
                      gx_v, ox_v, otail_v, in_sem, out_sem):
        wid = lax.axis_index("s") * nc + lax.axis_index("c")
        g_bufs = (g_v0, g_v1)
        o_bufs = (o_v0, o_v1)
        lanes = lax.iota(jnp.int32, _L)
        rots = [(lanes + r) & (_L - 1) for r in range(_L)]

        def in_start(ch, b):
            pltpu.async_copy(tt_hbm.at[:, pl.ds(ch * cw, cw)], g_bufs[b],
                             in_sem)

        def in_wait(b):
            pltpu.make_async_copy(tt_hbm.at[:, pl.ds(0, cw)], g_bufs[b],
                                  in_sem).wait()

        def out_start(ch, b):
            pltpu.async_copy(
                o_bufs[b], out_hbm.at[pl.ds(ch * (cw // 8), cw // 8)],
                out_sem)

        def out_wait(b):
            pltpu.make_async_copy(
                o_bufs[b], out_hbm.at[pl.ds(0, cw // 8)], out_sem).wait()

        def transpose_chunk(b):
            g_v, o_v = g_bufs[b], o_bufs[b]

            def body(m, carry):
                tv = m * _L + lanes
                orow = tv >> 3
                ocol = (tv & 7) << 4
                for r in range(_L):
                    vals = plsc.load_gather(g_v, [rots[r], tv])
                    plsc.store_scatter(o_v, [orow, ocol + rots[r]], vals)
                return carry

            lax.fori_loop(0, cw // _L, body, 0)

        base = wid * per_w
        in_start(base, 0)
        if per_w > 1:
            in_start(base + 1, 1)

        for i in range(min(2, per_w)):
            b = i % 2
            in_wait(b)
            transpose_chunk(b)
            out_start(base + i, b)
            if i + 2 < per_w:
                in_start(base + i + 2, b)

        assert per_w >= 7 and per_w % 2 == 1

        def pair_pf(io, carry):
            for b in range(2):
                i = io * 2 + b
                in_wait(b)
                out_wait(b)
                transpose_chunk(b)
                out_start(base + i, b)
                in_start(base + i + 2, b)
            return carry

        lax.fori_loop(1, (per_w - 3) // 2, pair_pf, 0)
        for i in range(per_w - 3, per_w):
            b = i % 2
            in_wait(b)
            out_wait(b)
            transpose_chunk(b)
            out_start(base + i, b)
            if i + 2 < per_w:
                in_start(base + i + 2, b)
        for b in range(2):
            out_wait(b)

        nchunks_total = nb_full // bpc
        if extra:
            @pl.when(wid < extra)
            def _():
                blk = nchunks_total * bpc + wid
                pltpu.async_copy(
                    tt_hbm.at[:, pl.ds(blk * 128, 128)], gx_v, in_sem)
                pltpu.make_async_copy(
                    tt_hbm.at[:, pl.ds(0, 128)], gx_v, in_sem).wait()

                def bbody(m, carry):
                    tv = m * _L + lanes
                    orow = tv >> 3
                    ocol = (tv & 7) << 4
                    for r in range(_L):
                        vals = plsc.load_gather(gx_v, [rots[r], tv])
                        plsc.store_scatter(
                            ox_v, [orow, ocol + rots[r]], vals)
                    return carry

                lax.fori_loop(0, 128 // _L, bbody, 0)
                pltpu.async_copy(
                    ox_v, out_hbm.at[pl.ds(blk * 16, 16)], out_sem)
                pltpu.make_async_copy(
                    ox_v, out_hbm.at[pl.ds(0, 16)], out_sem).wait()

        if tail:
            @pl.when(wid == extra)
            def _():
                pltpu.async_copy(tail_hbm, otail_v, in_sem)
                pltpu.make_async_copy(tail_hbm, otail_v, in_sem).wait()
                pltpu.async_copy(
                    otail_v,
                    out_hbm.at[pl.ds(nb_full * 16, tail * d // 128)], out_sem)
                pltpu.make_async_copy(
                    otail_v,
                    out_hbm.at[pl.ds(0, tail * d // 128)], out_sem).wait()

    return detile_kernel


def kernel(x, table):
    r, c = x.shape
    vocab, d = table.shape
    xt = jnp.transpose(x)
    nb_full = vocab // 128
    tail8 = lax.slice(table, (nb_full * 128, 0), (vocab, d)).reshape(
        (vocab - nb_full * 128) * d // 128, 128)
    tlin = _make_detile(vocab, d)(jnp.transpose(table), tail8).reshape(
        vocab, d)
    out5 = _make_gather(r, c, vocab, d)(xt, tlin)
    return jnp.transpose(out5, (2, 4, 0, 1, 3)).reshape(r, c, d)
